# baseline (speedup 1.0000x reference)
import functools

import jax
import jax.numpy as jnp
from jax import lax
from jax.experimental import pallas as pl
from jax.experimental.pallas import tpu as pltpu
from jax.experimental.pallas import tpu_sc as plsc

NU = 5000
NI = 5000
NN = NU + NI
D = 128
NH = 5120
NE = 160000
CHUNK = 128
NCH = 80
EPW = NCH * CHUNK
EPAD = 16 * EPW
RPT = NH // 16

_mesh = plsc.VectorSubcoreMesh(core_axis_name="c", subcore_axis_name="s")


def _deg_body(dst_hbm, out_hbm, idx_v, deg_v, sem):
    cid = lax.axis_index("c")
    sid = lax.axis_index("s")

    zeros16 = jnp.zeros((16,), jnp.float32)

    def zero_body(i, _):
        deg_v[pl.ds(i * 16, 16)] = zeros16
        return ()
    lax.fori_loop(0, NH // 16, zero_body, ())

    pltpu.sync_copy(dst_hbm.at[cid, sid], idx_v)

    ones16 = jnp.ones((16,), jnp.float32)

    def body(k, _):
        idx16 = idx_v[pl.ds(k * 16, 16)]
        plsc.addupdate_scatter(deg_v, [idx16], ones16)
        return ()
    lax.fori_loop(0, EPW // 16, body, ())

    pltpu.sync_copy(deg_v, out_hbm.at[cid, sid])


_deg_kernel = functools.partial(
    pl.kernel,
    out_type=jax.ShapeDtypeStruct((2, 16, NH), jnp.float32),
    mesh=_mesh,
    compiler_params=pltpu.CompilerParams(needs_layout_passes=False),
    scratch_types=[
        pltpu.VMEM((EPW,), jnp.int32),
        pltpu.VMEM((NH,), jnp.float32),
        pltpu.SemaphoreType.DMA,
    ],
)(_deg_body)


def _spmm_body(y_hbm, src_hbm, dst_hbm, out_hbm, srcv, dstv, rows0, rows1,
               acc_sh, sem0, sem1):
    cid = lax.axis_index("c")
    sid = lax.axis_index("s")

    pltpu.sync_copy(src_hbm.at[cid, sid], srcv)
    pltpu.sync_copy(dst_hbm.at[cid, sid], dstv)

    zeros16 = jnp.zeros((16,), jnp.float32)

    def zero_body(k, _):
        r = k // (D // 16)
        c = k % (D // 16)
        rows0[r, pl.ds(c * 16, 16)] = zeros16
        return ()
    lax.fori_loop(0, CHUNK * (D // 16), zero_body, ())

    def zcopy(b, _):
        pltpu.sync_copy(rows0, acc_sh.at[pl.ds(sid * RPT + b * CHUNK, CHUNK)])
        return ()
    lax.fori_loop(0, RPT // CHUNK, zcopy, ())
    pltpu.sync_copy(rows0.at[pl.ds(0, RPT % CHUNK)],
                    acc_sh.at[pl.ds(sid * RPT + (RPT // CHUNK) * CHUNK,
                                    RPT % CHUNK)])
    plsc.subcore_barrier()

    pltpu.async_copy(y_hbm.at[srcv.at[0]], rows0, sem0)
    pltpu.async_copy(y_hbm.at[srcv.at[1]], rows1, sem1)

    def body(g, _):
        j = 2 * g
        pltpu.make_async_copy(y_hbm.at[srcv.at[j]], rows0, sem0).wait()
        pltpu.sync_copy(rows0, acc_sh.at[dstv.at[j]], add=True)

        @pl.when(j + 2 < NCH)
        def _():
            pltpu.async_copy(y_hbm.at[srcv.at[j + 2]], rows0, sem0)

        pltpu.make_async_copy(y_hbm.at[srcv.at[j + 1]], rows1, sem1).wait()
        pltpu.sync_copy(rows1, acc_sh.at[dstv.at[j + 1]], add=True)

        @pl.when(j + 3 < NCH)
        def _():
            pltpu.async_copy(y_hbm.at[srcv.at[j + 3]], rows1, sem1)
        return ()
    lax.fori_loop(0, NCH // 2, body, ())

    plsc.subcore_barrier()
    pltpu.sync_copy(acc_sh.at[pl.ds(sid * RPT, RPT)],
                    out_hbm.at[cid, pl.ds(sid * RPT, RPT)])


_spmm_kernel = functools.partial(
    pl.kernel,
    out_type=jax.ShapeDtypeStruct((2, NH, D), jnp.float32),
    mesh=_mesh,
    scratch_types=[
        pltpu.VMEM((NCH, CHUNK), jnp.int32),
        pltpu.VMEM((NCH, CHUNK), jnp.int32),
        pltpu.VMEM((CHUNK, D), jnp.float32),
        pltpu.VMEM((CHUNK, D), jnp.float32),
        pltpu.VMEM_SHARED((NH, D), jnp.float32),
        pltpu.SemaphoreType.DMA,
        pltpu.SemaphoreType.DMA,
    ],
)(_spmm_body)


BLK = 1000


def _tc_layer_body(x_ref, acc_ref, db_ref, wu_ref, wi_ref,
                   msg_ref, int_ref, xn_ref, yn_ref):
    i = pl.program_id(0)
    x = x_ref[...]
    db = db_ref[...]
    msg = acc_ref[0] * db
    w = jnp.where(i < NU // BLK, wu_ref[...], wi_ref[...])
    logits = jnp.dot(x, w, preferred_element_type=jnp.float32)
    m = jnp.max(logits, axis=1, keepdims=True)
    e = jnp.exp(logits - m)
    p = e / jnp.sum(e, axis=1, keepdims=True)
    itl = lax.dot_general(p, w, (((1,), (1,)), ((), ())),
                          preferred_element_type=jnp.float32)
    msg_ref[...] = msg
    int_ref[...] = itl
    xn = msg + itl + x
    xn_ref[...] = xn
    yn_ref[...] = xn * db


def _tc_layer(x, acc, disb, wu, wi):
    nb = NU // BLK
    grid = (NN // BLK,)
    row_spec = pl.BlockSpec((BLK, D), lambda i: (i, 0))
    acc_spec = pl.BlockSpec((1, BLK, D), lambda i: (i // nb, i % nb, 0))
    w_spec = pl.BlockSpec((D, D), lambda i: (0, 0))
    out_sds = jax.ShapeDtypeStruct((NN, D), jnp.float32)
    return pl.pallas_call(
        _tc_layer_body,
        grid=grid,
        in_specs=[row_spec, acc_spec, row_spec, w_spec, w_spec],
        out_specs=[row_spec, row_spec, row_spec, row_spec],
        out_shape=[out_sds, out_sds, out_sds, out_sds],
    )(x, acc, disb, wu, wi)


def kernel(user_emb, item_emb, edge_index, user_intent, item_intent):
    h = edge_index[0].astype(jnp.int32)
    t = edge_index[1].astype(jnp.int32)
    npad = EPAD - NE
    pad0 = jnp.zeros((npad,), jnp.int32)
    padd = jnp.full((npad,), NU, jnp.int32)
    src = jnp.stack([jnp.concatenate([t + NU, pad0]),
                     jnp.concatenate([h, pad0])])
    dst = jnp.stack([jnp.concatenate([h, padd]),
                     jnp.concatenate([t, padd])])
    src4 = src.reshape(2, 16, NCH, CHUNK)
    dst4 = dst.reshape(2, 16, NCH, CHUNK)
    dst3 = dst.reshape(2, 16, EPW)

    degp = _deg_kernel(dst3)
    deg = jnp.concatenate([jnp.sum(degp[0], axis=0)[:NU],
                           jnp.sum(degp[1], axis=0)[:NI]])
    dis = jnp.where(deg > 0, lax.rsqrt(jnp.maximum(deg, 1.0)), 0.0)
    disb = jnp.broadcast_to(dis[:, None], (NN, D))

    e0 = jnp.concatenate([user_emb, item_emb], axis=0)
    y0 = e0 * disb

    acc0 = _spmm_kernel(y0, src4, dst4)
    msg0, int0, e1, y1 = _tc_layer(e0, acc0, disb, user_intent, item_intent)

    acc1 = _spmm_kernel(y1, src4, dst4)
    msg1, int1, e2, _ = _tc_layer(e1, acc1, disb, user_intent, item_intent)

    final = e0 + e1 + e2
    return (final[:NU], final[NU:],
            jnp.stack([msg0, msg1], axis=0),
            jnp.stack([int0, int1], axis=0))
